# baseline (speedup 1.0000x reference)
import functools

import jax
import jax.numpy as jnp
from jax import lax
from jax.experimental import pallas as pl
from jax.experimental.pallas import tpu as pltpu
from jax.experimental.pallas import tpu_sc as plsc

NUM_CLASSES = 21
ALPHA = 1.0
NEG_POS_RATIO = 3.0
NEGATIVES_FOR_HARD = 100.0

MBLK = 8192
NB = 1024
LANES = 16
NSUB = 16
WIN = 8192


def _tc_body(n_boxes, batches, yt_ref, conf_ref, loc_ref,
             conf_out_ref, score_out_ref, meta_ref, np_smem, acc_smem):
    i = pl.program_id(0)
    ng = pl.num_programs(0)

    @pl.when(i == 0)
    def _init():
        def zb(b, _):
            np_smem[b] = 0.0
            return 0
        lax.fori_loop(0, batches, zb, 0)
        acc_smem[0] = 0.0
        acc_smem[1] = 0.0
        meta_ref[...] = jnp.zeros((8, 128), jnp.float32)

    yt_ref = yt_ref[...].T
    x = conf_ref[...].T
    loc_t = loc_ref[...].T
    e = jnp.exp(x)
    s1 = jnp.sum(e, axis=0, keepdims=True)
    lse = jnp.log(s1)
    u = yt_ref[4:25, :]
    s2 = jnp.sum(u, axis=0, keepdims=True)
    s3 = jnp.sum(u * x, axis=0, keepdims=True)
    conf_loss = s2 * lse - s3
    mask = yt_ref[25:26, :]
    score = (1.0 - e[0:1, :] / s1) * (1.0 - mask)
    d = yt_ref[0:4, :] - loc_t
    ad = jnp.abs(d)
    ll = jnp.where(ad < 1.0, 0.5 * d * d, ad - 0.5)
    loc_loss = jnp.sum(ll, axis=0, keepdims=True)

    conf_out_ref[...] = conf_loss.reshape(1, 1, MBLK)
    score_out_ref[...] = score.reshape(1, 1, MBLK)

    b = i // (n_boxes // MBLK)
    np_smem[b] = np_smem[b] + jnp.sum(mask)
    acc_smem[0] = acc_smem[0] + jnp.sum(conf_loss * mask)
    acc_smem[1] = acc_smem[1] + jnp.sum(loc_loss * mask)

    @pl.when(i == ng - 1)
    def _fin():
        def body(bb, carry):
            nn_sum, has_min, denom = carry
            npb = np_smem[bb]
            nn = jnp.minimum(NEG_POS_RATIO * npb, n_boxes - npb)
            return (nn_sum + nn,
                    has_min + jnp.where(nn > 0.0, 1.0, 0.0),
                    denom + jnp.where(npb != 0.0, npb, 1.0))
        nn_sum, has_min, denom = lax.fori_loop(
            0, batches, body, (jnp.float32(0), jnp.float32(0), jnp.float32(0)))
        nnb = jnp.where(has_min > 0.0, nn_sum, jnp.float32(NEGATIVES_FOR_HARD))
        kneg = jnp.floor(nnb)
        pos_total = acc_smem[0] + ALPHA * acc_smem[1]
        meta_ref[...] = jnp.concatenate([
            jnp.full((1, 128), kneg, jnp.float32),
            jnp.full((1, 128), pos_total, jnp.float32),
            jnp.full((1, 128), denom, jnp.float32),
            jnp.zeros((5, 128), jnp.float32),
        ], axis=0)


def _tc_call(yt2, conf2, loc2, n_boxes, batches):
    f = yt2.shape[0]
    g = f // MBLK
    body = functools.partial(_tc_body, n_boxes, batches)
    return pl.pallas_call(
        body,
        grid=(g,),
        in_specs=[
            pl.BlockSpec((MBLK, 26), lambda i: (i, 0)),
            pl.BlockSpec((MBLK, NUM_CLASSES), lambda i: (i, 0)),
            pl.BlockSpec((MBLK, 4), lambda i: (i, 0)),
        ],
        out_specs=[
            pl.BlockSpec((1, 1, MBLK), lambda i: (i, 0, 0)),
            pl.BlockSpec((1, 1, MBLK), lambda i: (i, 0, 0)),
            pl.BlockSpec((8, 128), lambda i: (0, 0)),
        ],
        out_shape=[
            jax.ShapeDtypeStruct((g, 1, MBLK), jnp.float32),
            jax.ShapeDtypeStruct((g, 1, MBLK), jnp.float32),
            jax.ShapeDtypeStruct((8, 128), jnp.float32),
        ],
        scratch_shapes=[
            pltpu.SMEM((batches,), jnp.float32),
            pltpu.SMEM((2,), jnp.float32),
        ],
    )(yt2, conf2, loc2)


def _sc_body(f_total, score_hbm, conf_hbm, meta_hbm, out_hbm, stage_hbm,
             hcnt, hsum, sw, cw, red, tmp):
    s = lax.axis_index("s")
    lane = lax.iota(jnp.int32, 16)
    zero16 = jnp.zeros((16,), jnp.float32)
    ones16 = jnp.ones((16,), jnp.float32)

    def zbody(i, _):
        hcnt[pl.ds(i * 16, 16)] = zero16
        hsum[pl.ds(i * 16, 16)] = zero16
        return 0
    lax.fori_loop(0, LANES * NB // 16, zbody, 0)

    per = f_total // NSUB
    base = s * per

    def wbody(w, _):
        off = base + w * WIN
        pltpu.sync_copy(score_hbm.at[pl.ds(off, WIN)], sw)
        pltpu.sync_copy(conf_hbm.at[pl.ds(off, WIN)], cw)

        def ibody(j, _):
            v = sw[pl.ds(j * 16, 16)]
            cl = cw[pl.ds(j * 16, 16)]
            bkt = (v * jnp.float32(NB)).astype(jnp.int32)
            bkt = jnp.minimum(jnp.maximum(bkt, 0), NB - 1)
            addr = lane * NB + bkt
            plsc.addupdate_scatter(hcnt, [addr], ones16)
            plsc.addupdate_scatter(hsum, [addr], cl)
            return 0
        lax.fori_loop(0, WIN // 16, ibody, 0)
        return 0
    lax.fori_loop(0, per // WIN, wbody, 0)

    def rbody(k, _):
        acc_c = zero16
        acc_s = zero16
        for l in range(LANES):
            acc_c = acc_c + hcnt[pl.ds(l * NB + k * 16, 16)]
            acc_s = acc_s + hsum[pl.ds(l * NB + k * 16, 16)]
        red[pl.ds(k * 16, 16)] = acc_c
        red[pl.ds(NB + k * 16, 16)] = acc_s
        return 0
    lax.fori_loop(0, NB // 16, rbody, 0)

    pltpu.sync_copy(red, stage_hbm.at[pl.ds(s * 2 * NB, 2 * NB)])
    plsc.subcore_barrier()

    @pl.when(s == 0)
    def _finalize():
        def z2(i, _):
            red[pl.ds(i * 16, 16)] = zero16
            return 0
        lax.fori_loop(0, 2 * NB // 16, z2, 0)

        def mbody(t, _):
            pltpu.sync_copy(stage_hbm.at[pl.ds(t * 2 * NB, 2 * NB)], tmp)

            def abody(k, _):
                red[pl.ds(k * 16, 16)] = red[pl.ds(k * 16, 16)] + tmp[pl.ds(k * 16, 16)]
                return 0
            lax.fori_loop(0, 2 * NB // 16, abody, 0)
            return 0
        lax.fori_loop(0, NSUB, mbody, 0)

        pltpu.sync_copy(meta_hbm, tmp.at[pl.ds(0, 1024)])
        k16 = tmp[pl.ds(0, 16)]
        pos16 = tmp[pl.ds(128, 16)]
        den16 = tmp[pl.ds(256, 16)]

        ftot = jnp.float32(f_total)

        def scbody(k, carry):
            cc, cs, ak, ats, asum, ac = carry
            c16 = red[pl.ds(k * 16, 16)]
            s16 = red[pl.ds(NB + k * 16, 16)]
            t = cc + plsc.cumsum(c16)
            ts = cs + plsc.cumsum(s16)
            above = ftot - t
            sel = jnp.logical_and(above < k16, above + c16 >= k16)
            ak = ak + jnp.where(sel, k16 - above, 0.0)
            ats = ats + jnp.where(sel, ts, 0.0)
            asum = asum + jnp.where(sel, s16, 0.0)
            ac = ac + jnp.where(sel, c16, 0.0)
            cc = cc + jnp.broadcast_to(jnp.sum(c16), (16,))
            cs = cs + jnp.broadcast_to(jnp.sum(s16), (16,))
            return cc, cs, ak, ats, asum, ac

        init = (zero16, zero16, zero16, zero16, zero16, zero16)
        cc, cs, ak, ats, asum, ac = lax.fori_loop(0, NB // 16, scbody, init)

        r = jnp.broadcast_to(jnp.sum(ak), (16,))
        tsb = jnp.broadcast_to(jnp.sum(ats), (16,))
        sb = jnp.broadcast_to(jnp.sum(asum), (16,))
        cb = jnp.broadcast_to(jnp.sum(ac), (16,))
        neg = jnp.where(r > 0.0,
                        (cs - tsb) + r * sb / jnp.maximum(cb, 1.0),
                        zero16)
        loss16 = (pos16 + neg) / den16
        tmp[pl.ds(0, 16)] = loss16
        pltpu.sync_copy(tmp.at[pl.ds(0, 16)], out_hbm)


def _sc_call(score, conf, meta, f_total):
    mesh = plsc.VectorSubcoreMesh(
        core_axis_name="c", subcore_axis_name="s", num_cores=1)
    body = functools.partial(_sc_body, f_total)
    fn = pl.kernel(
        body,
        out_type=[
            jax.ShapeDtypeStruct((16,), jnp.float32),
            jax.ShapeDtypeStruct((NSUB * 2 * NB,), jnp.float32),
        ],
        mesh=mesh,
        scratch_types=[
            pltpu.VMEM((LANES * NB,), jnp.float32),
            pltpu.VMEM((LANES * NB,), jnp.float32),
            pltpu.VMEM((WIN,), jnp.float32),
            pltpu.VMEM((WIN,), jnp.float32),
            pltpu.VMEM((2 * NB,), jnp.float32),
            pltpu.VMEM((2 * NB,), jnp.float32),
        ],
        compiler_params=pltpu.CompilerParams(needs_layout_passes=False),
    )
    out16, _ = fn(score, conf, meta)
    return out16


def kernel(y_true, y_pred_loc, y_pred_conf):
    batches, n_boxes, _ = y_true.shape
    f_total = batches * n_boxes
    yt2 = y_true.reshape(f_total, 4 + NUM_CLASSES + 1)
    conf2 = y_pred_conf.reshape(f_total, NUM_CLASSES)
    loc2 = y_pred_loc.reshape(f_total, 4)
    conf3, score3, meta = _tc_call(yt2, conf2, loc2, n_boxes, batches)
    out16 = _sc_call(score3.reshape(f_total), conf3.reshape(f_total),
                     meta.reshape(1024), f_total)
    return out16[0]


# speedup vs baseline: 8.0002x; 8.0002x over previous
import functools

import jax
import jax.numpy as jnp
from jax import lax
from jax.experimental import pallas as pl
from jax.experimental.pallas import tpu as pltpu
from jax.experimental.pallas import tpu_sc as plsc

NUM_CLASSES = 21
ALPHA = 1.0
NEG_POS_RATIO = 3.0
NEGATIVES_FOR_HARD = 100.0

MBLK = 2048
NB = 1024
LANES = 16
NSUB = 16
WIN = 8192


def _tc_body(n_boxes, batches, yt_ref, conf_ref, loc_ref,
             conf_out_ref, score_out_ref, meta_ref, np_smem, acc_smem):
    g = pl.program_id(0)
    j = pl.program_id(1)
    is_first = jnp.logical_and(g == 0, j == 0)
    is_last = jnp.logical_and(g == pl.num_programs(0) - 1,
                              j == pl.num_programs(1) - 1)

    @pl.when(is_first)
    def _init():
        def zb(b, _):
            np_smem[b] = 0.0
            return 0
        lax.fori_loop(0, batches, zb, 0)
        acc_smem[0] = 0.0
        acc_smem[1] = 0.0
        meta_ref[...] = jnp.zeros((8, 128), jnp.float32)

    x = conf_ref[...]
    e = jnp.exp(x)
    s1 = jnp.sum(e, axis=0, keepdims=True)
    lse = jnp.log(s1)
    u = yt_ref[4:25]
    s2 = jnp.sum(u, axis=0, keepdims=True)
    s3 = jnp.sum(u * x, axis=0, keepdims=True)
    conf_loss = s2 * lse - s3
    mask = yt_ref[25:26]
    score = (1.0 - e[0:1] / s1) * (1.0 - mask)
    d = yt_ref[0:4] - loc_ref[...]
    ad = jnp.abs(d)
    ll = jnp.where(ad < 1.0, 0.5 * d * d, ad - 0.5)
    loc_loss = jnp.sum(ll, axis=0, keepdims=True)

    conf_out_ref[...] = conf_loss
    score_out_ref[...] = score

    for b in range(8):
        np_smem[g * 8 + b] = np_smem[g * 8 + b] + jnp.sum(mask[0, b, :])
    acc_smem[0] = acc_smem[0] + jnp.sum(conf_loss * mask)
    acc_smem[1] = acc_smem[1] + jnp.sum(loc_loss * mask)

    @pl.when(is_last)
    def _fin():
        def body(bb, carry):
            nn_sum, has_min, denom = carry
            npb = np_smem[bb]
            nn = jnp.minimum(NEG_POS_RATIO * npb, n_boxes - npb)
            return (nn_sum + nn,
                    has_min + jnp.where(nn > 0.0, 1.0, 0.0),
                    denom + jnp.where(npb != 0.0, npb, 1.0))
        nn_sum, has_min, denom = lax.fori_loop(
            0, batches, body, (jnp.float32(0), jnp.float32(0), jnp.float32(0)))
        nnb = jnp.where(has_min > 0.0, nn_sum, jnp.float32(NEGATIVES_FOR_HARD))
        kneg = jnp.floor(nnb)
        pos_total = acc_smem[0] + ALPHA * acc_smem[1]
        meta_ref[...] = jnp.concatenate([
            jnp.full((1, 128), kneg, jnp.float32),
            jnp.full((1, 128), pos_total, jnp.float32),
            jnp.full((1, 128), denom, jnp.float32),
            jnp.zeros((5, 128), jnp.float32),
        ], axis=0)


def _tc_call(ytt, conft, loct, n_boxes, batches):
    body = functools.partial(_tc_body, n_boxes, batches)
    return pl.pallas_call(
        body,
        grid=(batches // 8, n_boxes // MBLK),
        in_specs=[
            pl.BlockSpec((26, 8, MBLK), lambda g, j: (0, g, j)),
            pl.BlockSpec((NUM_CLASSES, 8, MBLK), lambda g, j: (0, g, j)),
            pl.BlockSpec((4, 8, MBLK), lambda g, j: (0, g, j)),
        ],
        out_specs=[
            pl.BlockSpec((1, 8, MBLK), lambda g, j: (0, g, j)),
            pl.BlockSpec((1, 8, MBLK), lambda g, j: (0, g, j)),
            pl.BlockSpec((8, 128), lambda g, j: (0, 0)),
        ],
        out_shape=[
            jax.ShapeDtypeStruct((1, batches, n_boxes), jnp.float32),
            jax.ShapeDtypeStruct((1, batches, n_boxes), jnp.float32),
            jax.ShapeDtypeStruct((8, 128), jnp.float32),
        ],
        scratch_shapes=[
            pltpu.SMEM((batches,), jnp.float32),
            pltpu.SMEM((2,), jnp.float32),
        ],
    )(ytt, conft, loct)


def _sc_body(f_total, score_hbm, conf_hbm, meta_hbm, out_hbm, stage_hbm,
             hcnt, hsum, sw, cw, red, tmp):
    s = lax.axis_index("s")
    lane = lax.iota(jnp.int32, 16)
    zero16 = jnp.zeros((16,), jnp.float32)
    ones16 = jnp.ones((16,), jnp.float32)

    def zbody(i, _):
        hcnt[pl.ds(i * 16, 16)] = zero16
        hsum[pl.ds(i * 16, 16)] = zero16
        return 0
    lax.fori_loop(0, LANES * NB // 16, zbody, 0)

    per = f_total // NSUB
    base = s * per

    def wbody(w, _):
        off = base + w * WIN
        pltpu.sync_copy(score_hbm.at[pl.ds(off, WIN)], sw)
        pltpu.sync_copy(conf_hbm.at[pl.ds(off, WIN)], cw)

        def ibody(j, _):
            v = sw[pl.ds(j * 16, 16)]
            cl = cw[pl.ds(j * 16, 16)]
            bkt = (v * jnp.float32(NB)).astype(jnp.int32)
            bkt = jnp.minimum(jnp.maximum(bkt, 0), NB - 1)
            addr = lane * NB + bkt
            plsc.addupdate_scatter(hcnt, [addr], ones16)
            plsc.addupdate_scatter(hsum, [addr], cl)
            return 0
        lax.fori_loop(0, WIN // 16, ibody, 0)
        return 0
    lax.fori_loop(0, per // WIN, wbody, 0)

    def rbody(k, _):
        acc_c = zero16
        acc_s = zero16
        for l in range(LANES):
            acc_c = acc_c + hcnt[pl.ds(l * NB + k * 16, 16)]
            acc_s = acc_s + hsum[pl.ds(l * NB + k * 16, 16)]
        red[pl.ds(k * 16, 16)] = acc_c
        red[pl.ds(NB + k * 16, 16)] = acc_s
        return 0
    lax.fori_loop(0, NB // 16, rbody, 0)

    pltpu.sync_copy(red, stage_hbm.at[pl.ds(s * 2 * NB, 2 * NB)])
    plsc.subcore_barrier()

    @pl.when(s == 0)
    def _finalize():
        def z2(i, _):
            red[pl.ds(i * 16, 16)] = zero16
            return 0
        lax.fori_loop(0, 2 * NB // 16, z2, 0)

        def mbody(t, _):
            pltpu.sync_copy(stage_hbm.at[pl.ds(t * 2 * NB, 2 * NB)], tmp)

            def abody(k, _):
                red[pl.ds(k * 16, 16)] = red[pl.ds(k * 16, 16)] + tmp[pl.ds(k * 16, 16)]
                return 0
            lax.fori_loop(0, 2 * NB // 16, abody, 0)
            return 0
        lax.fori_loop(0, NSUB, mbody, 0)

        pltpu.sync_copy(meta_hbm, tmp.at[pl.ds(0, 1024)])
        k16 = tmp[pl.ds(0, 16)]
        pos16 = tmp[pl.ds(128, 16)]
        den16 = tmp[pl.ds(256, 16)]

        ftot = jnp.float32(f_total)

        def scbody(k, carry):
            cc, cs, ak, ats, asum, ac = carry
            c16 = red[pl.ds(k * 16, 16)]
            s16 = red[pl.ds(NB + k * 16, 16)]
            t = cc + plsc.cumsum(c16)
            ts = cs + plsc.cumsum(s16)
            above = ftot - t
            sel = jnp.logical_and(above < k16, above + c16 >= k16)
            ak = ak + jnp.where(sel, k16 - above, 0.0)
            ats = ats + jnp.where(sel, ts, 0.0)
            asum = asum + jnp.where(sel, s16, 0.0)
            ac = ac + jnp.where(sel, c16, 0.0)
            cc = cc + jnp.broadcast_to(jnp.sum(c16), (16,))
            cs = cs + jnp.broadcast_to(jnp.sum(s16), (16,))
            return cc, cs, ak, ats, asum, ac

        init = (zero16, zero16, zero16, zero16, zero16, zero16)
        cc, cs, ak, ats, asum, ac = lax.fori_loop(0, NB // 16, scbody, init)

        r = jnp.broadcast_to(jnp.sum(ak), (16,))
        tsb = jnp.broadcast_to(jnp.sum(ats), (16,))
        sb = jnp.broadcast_to(jnp.sum(asum), (16,))
        cb = jnp.broadcast_to(jnp.sum(ac), (16,))
        neg = jnp.where(r > 0.0,
                        (cs - tsb) + r * sb / jnp.maximum(cb, 1.0),
                        zero16)
        loss16 = (pos16 + neg) / den16
        tmp[pl.ds(0, 16)] = loss16
        pltpu.sync_copy(tmp.at[pl.ds(0, 16)], out_hbm)


def _sc_call(score, conf, meta, f_total):
    mesh = plsc.VectorSubcoreMesh(
        core_axis_name="c", subcore_axis_name="s", num_cores=1)
    body = functools.partial(_sc_body, f_total)
    fn = pl.kernel(
        body,
        out_type=[
            jax.ShapeDtypeStruct((16,), jnp.float32),
            jax.ShapeDtypeStruct((NSUB * 2 * NB,), jnp.float32),
        ],
        mesh=mesh,
        scratch_types=[
            pltpu.VMEM((LANES * NB,), jnp.float32),
            pltpu.VMEM((LANES * NB,), jnp.float32),
            pltpu.VMEM((WIN,), jnp.float32),
            pltpu.VMEM((WIN,), jnp.float32),
            pltpu.VMEM((2 * NB,), jnp.float32),
            pltpu.VMEM((2 * NB,), jnp.float32),
        ],
        compiler_params=pltpu.CompilerParams(needs_layout_passes=False),
    )
    out16, _ = fn(score, conf, meta)
    return out16


def kernel(y_true, y_pred_loc, y_pred_conf):
    batches, n_boxes, _ = y_true.shape
    f_total = batches * n_boxes
    ytt = jnp.transpose(y_true, (2, 0, 1))
    conft = jnp.transpose(y_pred_conf, (2, 0, 1))
    loct = jnp.transpose(y_pred_loc, (2, 0, 1))
    conf3, score3, meta = _tc_call(ytt, conft, loct, n_boxes, batches)
    out16 = _sc_call(score3.reshape(f_total), conf3.reshape(f_total),
                     meta.reshape(1024), f_total)
    return out16[0]


# speedup vs baseline: 9.4571x; 1.1821x over previous
import functools

import jax
import jax.numpy as jnp
from jax import lax
from jax.experimental import pallas as pl
from jax.experimental.pallas import tpu as pltpu
from jax.experimental.pallas import tpu_sc as plsc

NUM_CLASSES = 21
ALPHA = 1.0
NEG_POS_RATIO = 3.0
NEGATIVES_FOR_HARD = 100.0

MBLK = 2048
NB = 1024
LANES = 16
NSUB = 16
WIN = 8192


def _tc_body(n_boxes, batches, yt_ref, conf_ref, loc_ref,
             conf_out_ref, score_out_ref, meta_ref, np_smem, acc_smem):
    g = pl.program_id(0)
    j = pl.program_id(1)
    is_first = jnp.logical_and(g == 0, j == 0)
    is_last = jnp.logical_and(g == pl.num_programs(0) - 1,
                              j == pl.num_programs(1) - 1)

    @pl.when(is_first)
    def _init():
        def zb(b, _):
            np_smem[b] = 0.0
            return 0
        lax.fori_loop(0, batches, zb, 0)
        acc_smem[0] = 0.0
        acc_smem[1] = 0.0
        meta_ref[...] = jnp.zeros((8, 128), jnp.float32)

    x = conf_ref[...]
    e = jnp.exp(x)
    s1 = jnp.sum(e, axis=0, keepdims=True)
    lse = jnp.log(s1)
    u = yt_ref[4:25]
    s2 = jnp.sum(u, axis=0, keepdims=True)
    s3 = jnp.sum(u * x, axis=0, keepdims=True)
    conf_loss = s2 * lse - s3
    mask = yt_ref[25:26]
    score = (1.0 - e[0:1] / s1) * (1.0 - mask)
    d = yt_ref[0:4] - loc_ref[...]
    ad = jnp.abs(d)
    ll = jnp.where(ad < 1.0, 0.5 * d * d, ad - 0.5)
    loc_loss = jnp.sum(ll, axis=0, keepdims=True)

    bkt = (score * jnp.float32(NB)).astype(jnp.int32)
    bkt = jnp.minimum(jnp.maximum(bkt, 0), NB - 1)
    lane16 = jax.lax.broadcasted_iota(jnp.int32, (1, 8, MBLK), 2) % 16
    addr = bkt + lane16 * NB

    def tiled(v):
        return jnp.transpose(v.reshape(8, MBLK // 128, 128), (1, 0, 2))[None]

    conf_out_ref[...] = tiled(conf_loss)
    score_out_ref[...] = tiled(addr)

    for b in range(8):
        np_smem[g * 8 + b] = np_smem[g * 8 + b] + jnp.sum(mask[0, b, :])
    acc_smem[0] = acc_smem[0] + jnp.sum(conf_loss * mask)
    acc_smem[1] = acc_smem[1] + jnp.sum(loc_loss * mask)

    @pl.when(is_last)
    def _fin():
        def body(bb, carry):
            nn_sum, has_min, denom = carry
            npb = np_smem[bb]
            nn = jnp.minimum(NEG_POS_RATIO * npb, n_boxes - npb)
            return (nn_sum + nn,
                    has_min + jnp.where(nn > 0.0, 1.0, 0.0),
                    denom + jnp.where(npb != 0.0, npb, 1.0))
        nn_sum, has_min, denom = lax.fori_loop(
            0, batches, body, (jnp.float32(0), jnp.float32(0), jnp.float32(0)))
        nnb = jnp.where(has_min > 0.0, nn_sum, jnp.float32(NEGATIVES_FOR_HARD))
        kneg = jnp.floor(nnb)
        pos_total = acc_smem[0] + ALPHA * acc_smem[1]
        meta_ref[...] = jnp.concatenate([
            jnp.full((1, 128), kneg, jnp.float32),
            jnp.full((1, 128), pos_total, jnp.float32),
            jnp.full((1, 128), denom, jnp.float32),
            jnp.zeros((5, 128), jnp.float32),
        ], axis=0)


def _tc_call(ytt, conft, loct, n_boxes, batches):
    body = functools.partial(_tc_body, n_boxes, batches)
    return pl.pallas_call(
        body,
        grid=(batches // 8, n_boxes // MBLK),
        in_specs=[
            pl.BlockSpec((26, 8, MBLK), lambda g, j: (0, g, j)),
            pl.BlockSpec((NUM_CLASSES, 8, MBLK), lambda g, j: (0, g, j)),
            pl.BlockSpec((4, 8, MBLK), lambda g, j: (0, g, j)),
        ],
        out_specs=[
            pl.BlockSpec((1, MBLK // 128, 8, 128), lambda g, j: (g, j, 0, 0)),
            pl.BlockSpec((1, MBLK // 128, 8, 128), lambda g, j: (g, j, 0, 0)),
            pl.BlockSpec((8, 128), lambda g, j: (0, 0)),
        ],
        out_shape=[
            jax.ShapeDtypeStruct((batches // 8, n_boxes // 128, 8, 128),
                                 jnp.float32),
            jax.ShapeDtypeStruct((batches // 8, n_boxes // 128, 8, 128),
                                 jnp.int32),
            jax.ShapeDtypeStruct((8, 128), jnp.float32),
        ],
        scratch_shapes=[
            pltpu.SMEM((batches,), jnp.float32),
            pltpu.SMEM((2,), jnp.float32),
        ],
    )(ytt, conft, loct)


def _sc_body(f_total, addr_hbm, conf_hbm, meta_hbm, out_hbm, stage_hbm,
             hcnt, hsum, sw, cw, red, tmp):
    s = lax.axis_index("s")
    zero16 = jnp.zeros((16,), jnp.float32)
    ones16 = jnp.ones((16,), jnp.float32)

    def zbody(i, _):
        hcnt[pl.ds(i * 16, 16)] = zero16
        hsum[pl.ds(i * 16, 16)] = zero16
        return 0
    lax.fori_loop(0, LANES * NB // 16, zbody, 0)

    per = f_total // NSUB
    base = s * per

    def wbody(w, _):
        off = base + w * WIN
        pltpu.sync_copy(addr_hbm.at[pl.ds(off, WIN)], sw)
        pltpu.sync_copy(conf_hbm.at[pl.ds(off, WIN)], cw)

        def ibody(j, _):
            a = sw[pl.ds(j * 16, 16)]
            cl = cw[pl.ds(j * 16, 16)]
            plsc.addupdate_scatter(hcnt, [a], ones16)
            plsc.addupdate_scatter(hsum, [a], cl)
            return 0
        lax.fori_loop(0, WIN // 16, ibody, 0)
        return 0
    lax.fori_loop(0, per // WIN, wbody, 0)

    def rbody(k, _):
        acc_c = zero16
        acc_s = zero16
        for l in range(LANES):
            acc_c = acc_c + hcnt[pl.ds(l * NB + k * 16, 16)]
            acc_s = acc_s + hsum[pl.ds(l * NB + k * 16, 16)]
        red[pl.ds(k * 16, 16)] = acc_c
        red[pl.ds(NB + k * 16, 16)] = acc_s
        return 0
    lax.fori_loop(0, NB // 16, rbody, 0)

    pltpu.sync_copy(red, stage_hbm.at[pl.ds(s * 2 * NB, 2 * NB)])
    plsc.subcore_barrier()

    @pl.when(s == 0)
    def _finalize():
        def z2(i, _):
            red[pl.ds(i * 16, 16)] = zero16
            return 0
        lax.fori_loop(0, 2 * NB // 16, z2, 0)

        def mbody(t, _):
            pltpu.sync_copy(stage_hbm.at[pl.ds(t * 2 * NB, 2 * NB)], tmp)

            def abody(k, _):
                red[pl.ds(k * 16, 16)] = red[pl.ds(k * 16, 16)] + tmp[pl.ds(k * 16, 16)]
                return 0
            lax.fori_loop(0, 2 * NB // 16, abody, 0)
            return 0
        lax.fori_loop(0, NSUB, mbody, 0)

        pltpu.sync_copy(meta_hbm, tmp.at[pl.ds(0, 1024)])
        k16 = tmp[pl.ds(0, 16)]
        pos16 = tmp[pl.ds(128, 16)]
        den16 = tmp[pl.ds(256, 16)]

        ftot = jnp.float32(f_total)

        def scbody(k, carry):
            cc, cs, ak, ats, asum, ac = carry
            c16 = red[pl.ds(k * 16, 16)]
            s16 = red[pl.ds(NB + k * 16, 16)]
            t = cc + plsc.cumsum(c16)
            ts = cs + plsc.cumsum(s16)
            above = ftot - t
            sel = jnp.logical_and(above < k16, above + c16 >= k16)
            ak = ak + jnp.where(sel, k16 - above, 0.0)
            ats = ats + jnp.where(sel, ts, 0.0)
            asum = asum + jnp.where(sel, s16, 0.0)
            ac = ac + jnp.where(sel, c16, 0.0)
            cc = cc + jnp.broadcast_to(jnp.sum(c16), (16,))
            cs = cs + jnp.broadcast_to(jnp.sum(s16), (16,))
            return cc, cs, ak, ats, asum, ac

        init = (zero16, zero16, zero16, zero16, zero16, zero16)
        cc, cs, ak, ats, asum, ac = lax.fori_loop(0, NB // 16, scbody, init)

        r = jnp.broadcast_to(jnp.sum(ak), (16,))
        tsb = jnp.broadcast_to(jnp.sum(ats), (16,))
        sb = jnp.broadcast_to(jnp.sum(asum), (16,))
        cb = jnp.broadcast_to(jnp.sum(ac), (16,))
        neg = jnp.where(r > 0.0,
                        (cs - tsb) + r * sb / jnp.maximum(cb, 1.0),
                        zero16)
        loss16 = (pos16 + neg) / den16
        tmp[pl.ds(0, 16)] = loss16
        pltpu.sync_copy(tmp.at[pl.ds(0, 16)], out_hbm)


def _sc_call(addr, conf, meta, f_total):
    mesh = plsc.VectorSubcoreMesh(
        core_axis_name="c", subcore_axis_name="s", num_cores=1)
    body = functools.partial(_sc_body, f_total)
    fn = pl.kernel(
        body,
        out_type=[
            jax.ShapeDtypeStruct((16,), jnp.float32),
            jax.ShapeDtypeStruct((NSUB * 2 * NB,), jnp.float32),
        ],
        mesh=mesh,
        scratch_types=[
            pltpu.VMEM((LANES * NB,), jnp.float32),
            pltpu.VMEM((LANES * NB,), jnp.float32),
            pltpu.VMEM((WIN,), jnp.int32),
            pltpu.VMEM((WIN,), jnp.float32),
            pltpu.VMEM((2 * NB,), jnp.float32),
            pltpu.VMEM((2 * NB,), jnp.float32),
        ],
        compiler_params=pltpu.CompilerParams(needs_layout_passes=False),
    )
    out16, _ = fn(addr, conf, meta)
    return out16


def kernel(y_true, y_pred_loc, y_pred_conf):
    batches, n_boxes, _ = y_true.shape
    f_total = batches * n_boxes
    ytt = jnp.transpose(y_true, (2, 0, 1))
    conft = jnp.transpose(y_pred_conf, (2, 0, 1))
    loct = jnp.transpose(y_pred_loc, (2, 0, 1))
    conf4, addr4, meta = _tc_call(ytt, conft, loct, n_boxes, batches)
    out16 = _sc_call(addr4.reshape(f_total), conf4.reshape(f_total),
                     meta.reshape(1024), f_total)
    return out16[0]
